# baseline (speedup 1.0000x reference)
import functools

import jax
import jax.numpy as jnp
from jax import lax
from jax.experimental import pallas as pl
from jax.experimental.pallas import tpu as pltpu
from jax.experimental.pallas import tpu_sc as plsc

_ROWS = 32768
_HID = 1024
_BLOCK = 2048

_NC = 2
_NS = 16
_NW = _NC * _NS
_WROWS = _ROWS // _NW
_BROWS = 32
_NDMA = _WROWS // _BROWS


def _logits_body(hs_ref, gw_ref, logits_ref):
    logits_ref[...] = jax.lax.dot_general(
        hs_ref[...], gw_ref[...],
        dimension_numbers=(((1,), (1,)), ((), ())),
        preferred_element_type=jnp.float32,
    )


def _zero_fill_body(out_hbm, buf, sem):
    wid = lax.axis_index("s") * _NC + lax.axis_index("c")
    base = wid * _WROWS

    def zero_buf(i, carry):
        r = i // (_HID // 16)
        c = i % (_HID // 16)
        buf[r, pl.ds(c * 16, 16)] = jnp.zeros((16,), jnp.float32)
        return carry

    lax.fori_loop(0, _BROWS * (_HID // 16), zero_buf, 0)

    def dma_ring(i, carry):
        pltpu.make_async_copy(
            buf, out_hbm.at[pl.ds(base + i * _BROWS, _BROWS), :], sem
        ).start()

        @pl.when(i >= 1)
        def _():
            pltpu.make_async_copy(
                buf, out_hbm.at[pl.ds(base + (i - 1) * _BROWS, _BROWS), :], sem
            ).wait()

        return carry

    lax.fori_loop(0, _NDMA, dma_ring, 0)
    pltpu.make_async_copy(
        buf, out_hbm.at[pl.ds(base + (_NDMA - 1) * _BROWS, _BROWS), :], sem
    ).wait()


_zero_fill = functools.partial(
    pl.kernel,
    out_type=jax.ShapeDtypeStruct((_ROWS, _HID), jnp.float32),
    mesh=plsc.VectorSubcoreMesh(core_axis_name="c", subcore_axis_name="s"),
    scratch_types=[
        pltpu.VMEM((_BROWS, _HID), jnp.float32),
        pltpu.SemaphoreType.DMA,
    ],
)(_zero_fill_body)


def kernel(hidden_states, gate_weight):
    batch, seq, hidden = hidden_states.shape
    rows = batch * seq
    hs = hidden_states.reshape(rows, hidden)
    num_experts = gate_weight.shape[0]

    logits = pl.pallas_call(
        _logits_body,
        grid=(rows // _BLOCK,),
        in_specs=[
            pl.BlockSpec((_BLOCK, hidden), lambda i: (i, 0)),
            pl.BlockSpec((num_experts, hidden), lambda i: (0, 0)),
        ],
        out_specs=pl.BlockSpec((_BLOCK, num_experts), lambda i: (i, 0)),
        out_shape=jax.ShapeDtypeStruct((rows, num_experts), jnp.float32),
    )(hs, gate_weight)

    zeros = _zero_fill()
    return zeros.reshape(batch, seq, hidden), logits


# speedup vs baseline: 1.0045x; 1.0045x over previous
import functools

import jax
import jax.numpy as jnp
from jax import lax
from jax.experimental import pallas as pl
from jax.experimental.pallas import tpu as pltpu
from jax.experimental.pallas import tpu_sc as plsc

_ROWS = 32768
_HID = 1024
_BLOCK = 2048

_NC = 2
_NS = 16
_NW = _NC * _NS
_WROWS = _ROWS // _NW
_BROWS = 32
_NDMA = _WROWS // _BROWS


def _logits_body(hs_ref, gw_ref, logits_ref):
    logits_ref[...] = jax.lax.dot_general(
        hs_ref[...], gw_ref[...],
        dimension_numbers=(((1,), (1,)), ((), ())),
        preferred_element_type=jnp.float32,
    )


def _zero_fill_body(out_hbm, buf, sem):
    wid = lax.axis_index("s") * _NC + lax.axis_index("c")
    base = wid * _WROWS

    def zero_row(r, carry):
        def zero_chunk(c, inner):
            buf[r, pl.ds(c * 16, 16)] = jnp.zeros((16,), jnp.float32)
            return inner

        return lax.fori_loop(0, _HID // 16, zero_chunk, carry)

    lax.fori_loop(0, _BROWS, zero_row, 0)

    def fire(i, carry):
        pltpu.make_async_copy(
            buf, out_hbm.at[pl.ds(base + i * _BROWS, _BROWS), :], sem
        ).start()
        return carry

    lax.fori_loop(0, _NDMA, fire, 0)

    def drain(i, carry):
        pltpu.make_async_copy(
            buf, out_hbm.at[pl.ds(base, _BROWS), :], sem
        ).wait()
        return carry

    lax.fori_loop(0, _NDMA, drain, 0)


_zero_fill = functools.partial(
    pl.kernel,
    out_type=jax.ShapeDtypeStruct((_ROWS, _HID), jnp.float32),
    mesh=plsc.VectorSubcoreMesh(core_axis_name="c", subcore_axis_name="s"),
    scratch_types=[
        pltpu.VMEM((_BROWS, _HID), jnp.float32),
        pltpu.SemaphoreType.DMA,
    ],
)(_zero_fill_body)


def kernel(hidden_states, gate_weight):
    batch, seq, hidden = hidden_states.shape
    rows = batch * seq
    hs = hidden_states.reshape(rows, hidden)
    num_experts = gate_weight.shape[0]

    zeros = _zero_fill()

    logits = pl.pallas_call(
        _logits_body,
        grid=(rows // _BLOCK,),
        in_specs=[
            pl.BlockSpec((_BLOCK, hidden), lambda i: (i, 0)),
            pl.BlockSpec((num_experts, hidden), lambda i: (0, 0)),
        ],
        out_specs=pl.BlockSpec((_BLOCK, num_experts), lambda i: (i, 0)),
        out_shape=jax.ShapeDtypeStruct((rows, num_experts), jnp.float32),
    )(hs, gate_weight)

    return zeros.reshape(batch, seq, hidden), logits
